# baseline (speedup 1.0000x reference)
import jax
import jax.numpy as jnp
from jax.experimental import pallas as pl

N = 10000
E = 320000


def _gcn_conv(x, src, dst, W, b, num_nodes):
    loop = jnp.arange(num_nodes, dtype=src.dtype)
    src_full = jnp.concatenate([src, loop])
    dst_full = jnp.concatenate([dst, loop])
    deg = jnp.zeros((num_nodes,), x.dtype).at[dst_full].add(1.0)
    dinv = jnp.where(deg > 0, 1.0 / jnp.sqrt(deg), 0.0)
    norm = dinv[src_full] * dinv[dst_full]
    h = x @ W
    msgs = h[src_full] * norm[:, None]
    out = jnp.zeros((num_nodes, W.shape[1]), x.dtype).at[dst_full].add(msgs)
    return out + b


def _final_body(h_ref, w_ref, b_ref, out_ref):
    out_ref[...] = h_ref[...] @ w_ref[...] + b_ref[...]


def kernel(x, edge_index, W1, b1, W2, b2, W3, b3, W4, b4, W5, b5, Wl, bl):
    src = edge_index[0]
    dst = edge_index[1]
    h = jnp.tanh(_gcn_conv(x, src, dst, W1, b1, N))
    h = jnp.tanh(_gcn_conv(h, src, dst, W2, b2, N))
    h = jnp.tanh(_gcn_conv(h, src, dst, W3, b3, N))
    h = jnp.tanh(_gcn_conv(h, src, dst, W4, b4, N))
    h = jnp.tanh(_gcn_conv(h, src, dst, W5, b5, N))
    output = pl.pallas_call(
        _final_body,
        out_shape=jax.ShapeDtypeStruct((N, Wl.shape[1]), x.dtype),
    )(h, Wl, bl[None, :])
    return (output, h)


# speedup vs baseline: 24.7717x; 24.7717x over previous
import functools

import jax
import jax.numpy as jnp
from jax import lax
from jax.experimental import pallas as pl
from jax.experimental.pallas import tpu as pltpu
from jax.experimental.pallas import tpu_sc as plsc

N = 10000
E = 320000
NP = 10240
NC, NS = 2, 16
NW = NC * NS
CH = 128
K = 79
EP = NW * K * CH
RPT = NP // NS
RB = 1024


def _sc_mesh():
    return plsc.VectorSubcoreMesh(core_axis_name="c", subcore_axis_name="s")


@functools.partial(
    pl.kernel,
    out_type=jax.ShapeDtypeStruct((NC, NP, 1), jnp.float32),
    mesh=_sc_mesh(),
    scratch_types=[
        pltpu.VMEM((K, CH), jnp.int32),
        pltpu.VMEM((CH, 1), jnp.float32),
        pltpu.VMEM_SHARED((NP, 1), jnp.float32),
    ],
    compiler_params=pltpu.CompilerParams(use_tc_tiling_on_sc=False),
)
def _sc_degree(ones_hbm, dstw_hbm, out_hbm, dst_v, ones_v, acc_s):
    c = lax.axis_index("c")
    s = lax.axis_index("s")
    wid = s * NC + c
    pltpu.sync_copy(dstw_hbm.at[wid], dst_v)
    pltpu.sync_copy(ones_hbm.at[pl.ds(0, CH)], ones_v)
    rs = s * RPT
    pltpu.sync_copy(ones_hbm.at[pl.ds(rs, RPT)], acc_s.at[pl.ds(rs, RPT)])
    plsc.subcore_barrier()

    def body(j, carry):
        pltpu.sync_copy(ones_v, acc_s.at[dst_v.at[j]], add=True)
        return carry

    lax.fori_loop(0, K, body, 0)
    plsc.subcore_barrier()
    pltpu.sync_copy(acc_s.at[pl.ds(rs, RPT)], out_hbm.at[c, pl.ds(rs, RPT)])


@functools.lru_cache(maxsize=None)
def _make_propagate(w):
    @functools.partial(
        pl.kernel,
        out_type=jax.ShapeDtypeStruct((NC, NP, w), jnp.float32),
        mesh=_sc_mesh(),
        scratch_types=[
            pltpu.VMEM((K, CH), jnp.int32),
            pltpu.VMEM((K, CH), jnp.int32),
            pltpu.VMEM((CH, w), jnp.float32),
            pltpu.VMEM_SHARED((NP, w), jnp.float32),
            pltpu.SemaphoreType.DMA,
        ],
        compiler_params=pltpu.CompilerParams(use_tc_tiling_on_sc=False),
    )
    def propagate(g_hbm, srcw_hbm, dstw_hbm, out_hbm, src_v, dst_v, rows_v,
                  acc_s, sem):
        c = lax.axis_index("c")
        s = lax.axis_index("s")
        wid = s * NC + c
        pltpu.sync_copy(srcw_hbm.at[wid], src_v)
        pltpu.sync_copy(dstw_hbm.at[wid], dst_v)
        rs = s * RPT
        pltpu.sync_copy(g_hbm.at[pl.ds(rs, RPT)], acc_s.at[pl.ds(rs, RPT)])
        plsc.subcore_barrier()

        def body(j, carry):
            pltpu.async_copy(g_hbm.at[src_v.at[j]], rows_v, sem).wait()
            pltpu.sync_copy(rows_v, acc_s.at[dst_v.at[j]], add=True)
            return carry

        lax.fori_loop(0, K, body, 0)
        plsc.subcore_barrier()
        pltpu.sync_copy(acc_s.at[pl.ds(rs, RPT)], out_hbm.at[c, pl.ds(rs, RPT)])

    return propagate


def _tc_head_body(dacc_ref, x_ref, w1_ref, dinv_ref, g_ref):
    deg = dacc_ref[0] + dacc_ref[1] - 1.0
    dinv = lax.rsqrt(deg)
    dinv_ref[...] = dinv
    g_ref[...] = jnp.dot(x_ref[...], w1_ref[...],
                         preferred_element_type=jnp.float32) * dinv


_tc_head = pl.pallas_call(
    _tc_head_body,
    grid=(NP // RB,),
    in_specs=[
        pl.BlockSpec((NC, RB, 1), lambda i: (0, i, 0)),
        pl.BlockSpec((RB, 128), lambda i: (i, 0)),
        pl.BlockSpec((128, 16), lambda i: (0, 0)),
    ],
    out_specs=[
        pl.BlockSpec((RB, 1), lambda i: (i, 0)),
        pl.BlockSpec((RB, 16), lambda i: (i, 0)),
    ],
    out_shape=[
        jax.ShapeDtypeStruct((NP, 1), jnp.float32),
        jax.ShapeDtypeStruct((NP, 16), jnp.float32),
    ],
)


def _tc_mid_body(acc_ref, g_ref, dinv_ref, wn_ref, b_ref, gn_ref):
    dinv = dinv_ref[...]
    act = jnp.tanh(dinv * (acc_ref[0] + acc_ref[1] - g_ref[...]) + b_ref[...])
    gn_ref[...] = jnp.dot(act, wn_ref[...],
                          preferred_element_type=jnp.float32) * dinv


@functools.lru_cache(maxsize=None)
def _make_tc_mid(w, w2):
    return pl.pallas_call(
        _tc_mid_body,
        grid=(NP // RB,),
        in_specs=[
            pl.BlockSpec((NC, RB, w), lambda i: (0, i, 0)),
            pl.BlockSpec((RB, w), lambda i: (i, 0)),
            pl.BlockSpec((RB, 1), lambda i: (i, 0)),
            pl.BlockSpec((w, w2), lambda i: (0, 0)),
            pl.BlockSpec((1, w), lambda i: (0, 0)),
        ],
        out_specs=pl.BlockSpec((RB, w2), lambda i: (i, 0)),
        out_shape=jax.ShapeDtypeStruct((NP, w2), jnp.float32),
    )


def _tc_tail_body(acc_ref, g_ref, dinv_ref, b5_ref, wl_ref, bl_ref,
                  out_ref, h_ref):
    act = jnp.tanh(dinv_ref[...] * (acc_ref[0] + acc_ref[1] - g_ref[...])
                   + b5_ref[...])
    h_ref[...] = act
    out_ref[...] = jnp.dot(act, wl_ref[...],
                           preferred_element_type=jnp.float32) + bl_ref[...]


_tc_tail = pl.pallas_call(
    _tc_tail_body,
    grid=(NP // RB,),
    in_specs=[
        pl.BlockSpec((NC, RB, 2), lambda i: (0, i, 0)),
        pl.BlockSpec((RB, 2), lambda i: (i, 0)),
        pl.BlockSpec((RB, 1), lambda i: (i, 0)),
        pl.BlockSpec((1, 2), lambda i: (0, 0)),
        pl.BlockSpec((2, 16), lambda i: (0, 0)),
        pl.BlockSpec((1, 16), lambda i: (0, 0)),
    ],
    out_specs=[
        pl.BlockSpec((RB, 16), lambda i: (i, 0)),
        pl.BlockSpec((RB, 2), lambda i: (i, 0)),
    ],
    out_shape=[
        jax.ShapeDtypeStruct((NP, 16), jnp.float32),
        jax.ShapeDtypeStruct((NP, 2), jnp.float32),
    ],
)


def kernel(x, edge_index, W1, b1, W2, b2, W3, b3, W4, b4, W5, b5, Wl, bl):
    src = edge_index[0].astype(jnp.int32)
    dst = edge_index[1].astype(jnp.int32)
    pad = jnp.full((EP - E,), N, jnp.int32)
    srcw = jnp.concatenate([src, pad]).reshape(NW, K, CH)
    dstw = jnp.concatenate([dst, pad]).reshape(NW, K, CH)
    xp = jnp.zeros((NP, 128), jnp.float32).at[:N].set(x)
    ones_col = jnp.ones((NP, 1), jnp.float32)

    dacc = _sc_degree(ones_col, dstw)
    dinv, g = _tc_head(dacc, xp, W1)

    acc = _make_propagate(16)(g, srcw, dstw)
    g = _make_tc_mid(16, 4)(acc, g, dinv, W2, b1[None, :])
    acc = _make_propagate(4)(g, srcw, dstw)
    g = _make_tc_mid(4, 4)(acc, g, dinv, W3, b2[None, :])
    acc = _make_propagate(4)(g, srcw, dstw)
    g = _make_tc_mid(4, 2)(acc, g, dinv, W4, b3[None, :])
    acc = _make_propagate(2)(g, srcw, dstw)
    g = _make_tc_mid(2, 2)(acc, g, dinv, W5, b4[None, :])
    acc = _make_propagate(2)(g, srcw, dstw)
    out, h = _tc_tail(acc, g, dinv, b5[None, :], Wl, bl[None, :])
    return (out[:N], h[:N])


# speedup vs baseline: 28.2940x; 1.1422x over previous
import functools

import jax
import jax.numpy as jnp
from jax import lax
from jax.experimental import pallas as pl
from jax.experimental.pallas import tpu as pltpu
from jax.experimental.pallas import tpu_sc as plsc

N = 10000
E = 320000
NP = 10240
NC, NS = 2, 16
NW = NC * NS
CH = 128
K = 80
B = 16
NB = K // B
EP = NW * K * CH
RPT = NP // NS
RB = 1024


def _sc_mesh():
    return plsc.VectorSubcoreMesh(core_axis_name="c", subcore_axis_name="s")


@functools.partial(
    pl.kernel,
    out_type=jax.ShapeDtypeStruct((NC, NP, 1), jnp.float32),
    mesh=_sc_mesh(),
    scratch_types=[
        pltpu.VMEM((K, CH), jnp.int32),
        pltpu.VMEM((CH, 1), jnp.float32),
        pltpu.VMEM_SHARED((NP, 1), jnp.float32),
        pltpu.SemaphoreType.DMA,
    ],
    compiler_params=pltpu.CompilerParams(use_tc_tiling_on_sc=False),
)
def _sc_degree(ones_hbm, dstw_hbm, out_hbm, dst_v, ones_v, acc_s, sem):
    c = lax.axis_index("c")
    s = lax.axis_index("s")
    wid = s * NC + c
    pltpu.sync_copy(dstw_hbm.at[wid], dst_v)
    pltpu.sync_copy(ones_hbm.at[pl.ds(0, CH)], ones_v)
    rs = s * RPT
    pltpu.sync_copy(ones_hbm.at[pl.ds(rs, RPT)], acc_s.at[pl.ds(rs, RPT)])
    plsc.subcore_barrier()

    def body(b, carry):
        ds = [pltpu.async_copy(ones_v, acc_s.at[dst_v.at[b * B + i]], sem,
                               add=True) for i in range(B)]
        for d in ds:
            d.wait()
        return carry

    lax.fori_loop(0, NB, body, 0)
    plsc.subcore_barrier()
    pltpu.sync_copy(acc_s.at[pl.ds(rs, RPT)], out_hbm.at[c, pl.ds(rs, RPT)])


@functools.lru_cache(maxsize=None)
def _make_propagate(w):
    @functools.partial(
        pl.kernel,
        out_type=jax.ShapeDtypeStruct((NC, NP, w), jnp.float32),
        mesh=_sc_mesh(),
        scratch_types=[
            pltpu.VMEM((K, CH), jnp.int32),
            pltpu.VMEM((K, CH), jnp.int32),
            pltpu.VMEM((B, CH, w), jnp.float32),
            pltpu.VMEM_SHARED((NP, w), jnp.float32),
            pltpu.SemaphoreType.DMA,
            pltpu.SemaphoreType.DMA,
        ],
        compiler_params=pltpu.CompilerParams(use_tc_tiling_on_sc=False),
    )
    def propagate(g_hbm, srcw_hbm, dstw_hbm, out_hbm, src_v, dst_v, rows_v,
                  acc_s, gsem, ssem):
        c = lax.axis_index("c")
        s = lax.axis_index("s")
        wid = s * NC + c
        pltpu.sync_copy(srcw_hbm.at[wid], src_v)
        pltpu.sync_copy(dstw_hbm.at[wid], dst_v)
        rs = s * RPT
        pltpu.sync_copy(g_hbm.at[pl.ds(rs, RPT)], acc_s.at[pl.ds(rs, RPT)])
        plsc.subcore_barrier()

        def body(b, carry):
            gs = [pltpu.async_copy(g_hbm.at[src_v.at[b * B + i]],
                                   rows_v.at[i], gsem) for i in range(B)]
            for d in gs:
                d.wait()
            ss = [pltpu.async_copy(rows_v.at[i], acc_s.at[dst_v.at[b * B + i]],
                                   ssem, add=True) for i in range(B)]
            for d in ss:
                d.wait()
            return carry

        lax.fori_loop(0, NB, body, 0)
        plsc.subcore_barrier()
        pltpu.sync_copy(acc_s.at[pl.ds(rs, RPT)], out_hbm.at[c, pl.ds(rs, RPT)])

    return propagate


def _tc_head_body(dacc_ref, x_ref, w1_ref, dinv_ref, g_ref):
    deg = dacc_ref[0] + dacc_ref[1] - 1.0
    dinv = lax.rsqrt(deg)
    dinv_ref[...] = dinv
    g_ref[...] = jnp.dot(x_ref[...], w1_ref[...],
                         preferred_element_type=jnp.float32) * dinv


_tc_head = pl.pallas_call(
    _tc_head_body,
    grid=(NP // RB,),
    in_specs=[
        pl.BlockSpec((NC, RB, 1), lambda i: (0, i, 0)),
        pl.BlockSpec((RB, 128), lambda i: (i, 0)),
        pl.BlockSpec((128, 16), lambda i: (0, 0)),
    ],
    out_specs=[
        pl.BlockSpec((RB, 1), lambda i: (i, 0)),
        pl.BlockSpec((RB, 16), lambda i: (i, 0)),
    ],
    out_shape=[
        jax.ShapeDtypeStruct((NP, 1), jnp.float32),
        jax.ShapeDtypeStruct((NP, 16), jnp.float32),
    ],
)


def _tc_mid_body(acc_ref, g_ref, dinv_ref, wn_ref, b_ref, gn_ref):
    dinv = dinv_ref[...]
    act = jnp.tanh(dinv * (acc_ref[0] + acc_ref[1] - g_ref[...]) + b_ref[...])
    gn_ref[...] = jnp.dot(act, wn_ref[...],
                          preferred_element_type=jnp.float32) * dinv


@functools.lru_cache(maxsize=None)
def _make_tc_mid(w, w2):
    return pl.pallas_call(
        _tc_mid_body,
        grid=(NP // RB,),
        in_specs=[
            pl.BlockSpec((NC, RB, w), lambda i: (0, i, 0)),
            pl.BlockSpec((RB, w), lambda i: (i, 0)),
            pl.BlockSpec((RB, 1), lambda i: (i, 0)),
            pl.BlockSpec((w, w2), lambda i: (0, 0)),
            pl.BlockSpec((1, w), lambda i: (0, 0)),
        ],
        out_specs=pl.BlockSpec((RB, w2), lambda i: (i, 0)),
        out_shape=jax.ShapeDtypeStruct((NP, w2), jnp.float32),
    )


def _tc_tail_body(acc_ref, g_ref, dinv_ref, b5_ref, wl_ref, bl_ref,
                  out_ref, h_ref):
    act = jnp.tanh(dinv_ref[...] * (acc_ref[0] + acc_ref[1] - g_ref[...])
                   + b5_ref[...])
    h_ref[...] = act
    out_ref[...] = jnp.dot(act, wl_ref[...],
                           preferred_element_type=jnp.float32) + bl_ref[...]


_tc_tail = pl.pallas_call(
    _tc_tail_body,
    grid=(NP // RB,),
    in_specs=[
        pl.BlockSpec((NC, RB, 2), lambda i: (0, i, 0)),
        pl.BlockSpec((RB, 2), lambda i: (i, 0)),
        pl.BlockSpec((RB, 1), lambda i: (i, 0)),
        pl.BlockSpec((1, 2), lambda i: (0, 0)),
        pl.BlockSpec((2, 16), lambda i: (0, 0)),
        pl.BlockSpec((1, 16), lambda i: (0, 0)),
    ],
    out_specs=[
        pl.BlockSpec((RB, 16), lambda i: (i, 0)),
        pl.BlockSpec((RB, 2), lambda i: (i, 0)),
    ],
    out_shape=[
        jax.ShapeDtypeStruct((NP, 16), jnp.float32),
        jax.ShapeDtypeStruct((NP, 2), jnp.float32),
    ],
)


def kernel(x, edge_index, W1, b1, W2, b2, W3, b3, W4, b4, W5, b5, Wl, bl):
    src = edge_index[0].astype(jnp.int32)
    dst = edge_index[1].astype(jnp.int32)
    pad = jnp.full((EP - E,), N, jnp.int32)
    srcw = jnp.concatenate([src, pad]).reshape(NW, K, CH)
    dstw = jnp.concatenate([dst, pad]).reshape(NW, K, CH)
    xp = jnp.zeros((NP, 128), jnp.float32).at[:N].set(x)
    ones_col = jnp.ones((NP, 1), jnp.float32)

    dacc = _sc_degree(ones_col, dstw)
    dinv, g = _tc_head(dacc, xp, W1)

    acc = _make_propagate(16)(g, srcw, dstw)
    g = _make_tc_mid(16, 4)(acc, g, dinv, W2, b1[None, :])
    acc = _make_propagate(4)(g, srcw, dstw)
    g = _make_tc_mid(4, 4)(acc, g, dinv, W3, b2[None, :])
    acc = _make_propagate(4)(g, srcw, dstw)
    g = _make_tc_mid(4, 2)(acc, g, dinv, W4, b3[None, :])
    acc = _make_propagate(2)(g, srcw, dstw)
    g = _make_tc_mid(2, 2)(acc, g, dinv, W5, b4[None, :])
    acc = _make_propagate(2)(g, srcw, dstw)
    out, h = _tc_tail(acc, g, dinv, b5[None, :], Wl, bl[None, :])
    return (out[:N], h[:N])


# speedup vs baseline: 44.2220x; 1.5629x over previous
import functools

import jax
import jax.numpy as jnp
from jax import lax
from jax.experimental import pallas as pl
from jax.experimental.pallas import tpu as pltpu
from jax.experimental.pallas import tpu_sc as plsc

N = 10000
E = 320000
NP = 10240
NC, NS = 2, 16
NW = NC * NS
CH = 128
K = 80
B = 16
NB = K // B
EP = NW * K * CH
RPT = NP // NS
RB = 1024


def _sc_mesh():
    return plsc.VectorSubcoreMesh(core_axis_name="c", subcore_axis_name="s")


@functools.partial(
    pl.kernel,
    out_type=jax.ShapeDtypeStruct((NC, NP, 1), jnp.float32),
    mesh=_sc_mesh(),
    scratch_types=[
        pltpu.VMEM((K, CH), jnp.int32),
        pltpu.VMEM((CH, 1), jnp.float32),
        pltpu.VMEM_SHARED((NP, 1), jnp.float32),
        pltpu.SemaphoreType.DMA,
    ],
    compiler_params=pltpu.CompilerParams(use_tc_tiling_on_sc=False),
)
def _sc_degree(ones_hbm, dstw_hbm, out_hbm, dst_v, ones_v, acc_s, sem):
    c = lax.axis_index("c")
    s = lax.axis_index("s")
    wid = s * NC + c
    pltpu.sync_copy(dstw_hbm.at[wid], dst_v)
    pltpu.sync_copy(ones_hbm.at[pl.ds(0, CH)], ones_v)
    rs = s * RPT
    pltpu.sync_copy(ones_hbm.at[pl.ds(rs, RPT)], acc_s.at[pl.ds(rs, RPT)])
    plsc.subcore_barrier()

    def body(b, carry):
        ds = [pltpu.async_copy(ones_v, acc_s.at[dst_v.at[b * B + i]], sem,
                               add=True) for i in range(B)]
        for d in ds:
            d.wait()
        return carry

    lax.fori_loop(0, NB, body, 0)
    plsc.subcore_barrier()
    pltpu.sync_copy(acc_s.at[pl.ds(rs, RPT)], out_hbm.at[c, pl.ds(rs, RPT)])


@functools.lru_cache(maxsize=None)
def _make_propagate(w):
    @functools.partial(
        pl.kernel,
        out_type=jax.ShapeDtypeStruct((NC, NP, w), jnp.float32),
        mesh=_sc_mesh(),
        scratch_types=[
            pltpu.VMEM((K, CH), jnp.int32),
            pltpu.VMEM((K, CH), jnp.int32),
            pltpu.VMEM((B, CH, w), jnp.float32),
            pltpu.VMEM_SHARED((NP, w), jnp.float32),
            pltpu.VMEM_SHARED((NP, w), jnp.float32),
            pltpu.SemaphoreType.DMA,
            pltpu.SemaphoreType.DMA,
        ],
        compiler_params=pltpu.CompilerParams(use_tc_tiling_on_sc=False),
    )
    def propagate(g_hbm, srcw_hbm, dstw_hbm, out_hbm, src_v, dst_v, rows_v,
                  table_s, acc_s, gsem, ssem):
        c = lax.axis_index("c")
        s = lax.axis_index("s")
        wid = s * NC + c
        pltpu.sync_copy(srcw_hbm.at[wid], src_v)
        pltpu.sync_copy(dstw_hbm.at[wid], dst_v)
        rs = s * RPT
        pltpu.sync_copy(g_hbm.at[pl.ds(rs, RPT)], table_s.at[pl.ds(rs, RPT)])
        pltpu.sync_copy(g_hbm.at[pl.ds(rs, RPT)], acc_s.at[pl.ds(rs, RPT)])
        plsc.subcore_barrier()

        def body(b, carry):
            gs = [pltpu.async_copy(table_s.at[src_v.at[b * B + i]],
                                   rows_v.at[i], gsem) for i in range(B)]
            for d in gs:
                d.wait()
            ss = [pltpu.async_copy(rows_v.at[i], acc_s.at[dst_v.at[b * B + i]],
                                   ssem, add=True) for i in range(B)]
            for d in ss:
                d.wait()
            return carry

        lax.fori_loop(0, NB, body, 0)
        plsc.subcore_barrier()
        pltpu.sync_copy(acc_s.at[pl.ds(rs, RPT)], out_hbm.at[c, pl.ds(rs, RPT)])

    return propagate


def _tc_head_body(dacc_ref, x_ref, w1_ref, dinv_ref, g_ref):
    deg = dacc_ref[0] + dacc_ref[1] - 1.0
    dinv = lax.rsqrt(deg)
    dinv_ref[...] = dinv
    g_ref[...] = jnp.dot(x_ref[...], w1_ref[...],
                         preferred_element_type=jnp.float32) * dinv


_tc_head = pl.pallas_call(
    _tc_head_body,
    grid=(NP // RB,),
    in_specs=[
        pl.BlockSpec((NC, RB, 1), lambda i: (0, i, 0)),
        pl.BlockSpec((RB, 128), lambda i: (i, 0)),
        pl.BlockSpec((128, 16), lambda i: (0, 0)),
    ],
    out_specs=[
        pl.BlockSpec((RB, 1), lambda i: (i, 0)),
        pl.BlockSpec((RB, 16), lambda i: (i, 0)),
    ],
    out_shape=[
        jax.ShapeDtypeStruct((NP, 1), jnp.float32),
        jax.ShapeDtypeStruct((NP, 16), jnp.float32),
    ],
)


def _tc_mid_body(acc_ref, g_ref, dinv_ref, wn_ref, b_ref, gn_ref):
    dinv = dinv_ref[...]
    act = jnp.tanh(dinv * (acc_ref[0] + acc_ref[1] - g_ref[...]) + b_ref[...])
    gn_ref[...] = jnp.dot(act, wn_ref[...],
                          preferred_element_type=jnp.float32) * dinv


@functools.lru_cache(maxsize=None)
def _make_tc_mid(w, w2):
    return pl.pallas_call(
        _tc_mid_body,
        grid=(NP // RB,),
        in_specs=[
            pl.BlockSpec((NC, RB, w), lambda i: (0, i, 0)),
            pl.BlockSpec((RB, w), lambda i: (i, 0)),
            pl.BlockSpec((RB, 1), lambda i: (i, 0)),
            pl.BlockSpec((w, w2), lambda i: (0, 0)),
            pl.BlockSpec((1, w), lambda i: (0, 0)),
        ],
        out_specs=pl.BlockSpec((RB, w2), lambda i: (i, 0)),
        out_shape=jax.ShapeDtypeStruct((NP, w2), jnp.float32),
    )


def _tc_tail_body(acc_ref, g_ref, dinv_ref, b5_ref, wl_ref, bl_ref,
                  out_ref, h_ref):
    act = jnp.tanh(dinv_ref[...] * (acc_ref[0] + acc_ref[1] - g_ref[...])
                   + b5_ref[...])
    h_ref[...] = act
    out_ref[...] = jnp.dot(act, wl_ref[...],
                           preferred_element_type=jnp.float32) + bl_ref[...]


_tc_tail = pl.pallas_call(
    _tc_tail_body,
    grid=(NP // RB,),
    in_specs=[
        pl.BlockSpec((NC, RB, 2), lambda i: (0, i, 0)),
        pl.BlockSpec((RB, 2), lambda i: (i, 0)),
        pl.BlockSpec((RB, 1), lambda i: (i, 0)),
        pl.BlockSpec((1, 2), lambda i: (0, 0)),
        pl.BlockSpec((2, 16), lambda i: (0, 0)),
        pl.BlockSpec((1, 16), lambda i: (0, 0)),
    ],
    out_specs=[
        pl.BlockSpec((RB, 16), lambda i: (i, 0)),
        pl.BlockSpec((RB, 2), lambda i: (i, 0)),
    ],
    out_shape=[
        jax.ShapeDtypeStruct((NP, 16), jnp.float32),
        jax.ShapeDtypeStruct((NP, 2), jnp.float32),
    ],
)


def kernel(x, edge_index, W1, b1, W2, b2, W3, b3, W4, b4, W5, b5, Wl, bl):
    src = edge_index[0].astype(jnp.int32)
    dst = edge_index[1].astype(jnp.int32)
    pad = jnp.full((EP - E,), N, jnp.int32)
    srcw = jnp.concatenate([src, pad]).reshape(NW, K, CH)
    dstw = jnp.concatenate([dst, pad]).reshape(NW, K, CH)
    xp = jnp.zeros((NP, 128), jnp.float32).at[:N].set(x)
    ones_col = jnp.ones((NP, 1), jnp.float32)

    dacc = _sc_degree(ones_col, dstw)
    dinv, g = _tc_head(dacc, xp, W1)

    acc = _make_propagate(16)(g, srcw, dstw)
    g = _make_tc_mid(16, 4)(acc, g, dinv, W2, b1[None, :])
    acc = _make_propagate(4)(g, srcw, dstw)
    g = _make_tc_mid(4, 4)(acc, g, dinv, W3, b2[None, :])
    acc = _make_propagate(4)(g, srcw, dstw)
    g = _make_tc_mid(4, 2)(acc, g, dinv, W4, b3[None, :])
    acc = _make_propagate(2)(g, srcw, dstw)
    g = _make_tc_mid(2, 2)(acc, g, dinv, W5, b4[None, :])
    acc = _make_propagate(2)(g, srcw, dstw)
    out, h = _tc_tail(acc, g, dinv, b5[None, :], Wl, bl[None, :])
    return (out[:N], h[:N])


# speedup vs baseline: 46.1757x; 1.0442x over previous
import functools

import jax
import jax.numpy as jnp
from jax import lax
from jax.experimental import pallas as pl
from jax.experimental.pallas import tpu as pltpu
from jax.experimental.pallas import tpu_sc as plsc

N = 10000
E = 320000
NP = 10240
NC, NS = 2, 16
NW = NC * NS
CH = 128
K = 80
B = 16
NB = K // B
EP = NW * K * CH
RPT = NP // NS
RB = 1024


def _sc_mesh():
    return plsc.VectorSubcoreMesh(core_axis_name="c", subcore_axis_name="s")


@functools.partial(
    pl.kernel,
    out_type=jax.ShapeDtypeStruct((NC, NP, 1), jnp.float32),
    mesh=_sc_mesh(),
    scratch_types=[
        pltpu.VMEM((K, CH), jnp.int32),
        pltpu.VMEM((CH, 1), jnp.float32),
        pltpu.VMEM_SHARED((NP, 1), jnp.float32),
        pltpu.SemaphoreType.DMA,
    ],
    compiler_params=pltpu.CompilerParams(use_tc_tiling_on_sc=False),
)
def _sc_degree(ones_hbm, dstw_hbm, out_hbm, dst_v, ones_v, acc_s, sem):
    c = lax.axis_index("c")
    s = lax.axis_index("s")
    wid = s * NC + c
    pltpu.sync_copy(dstw_hbm.at[wid], dst_v)
    pltpu.sync_copy(ones_hbm.at[pl.ds(0, CH)], ones_v)
    rs = s * RPT
    pltpu.sync_copy(ones_hbm.at[pl.ds(rs, RPT)], acc_s.at[pl.ds(rs, RPT)])
    plsc.subcore_barrier()

    def fire(b, carry):
        for i in range(B):
            pltpu.async_copy(ones_v, acc_s.at[dst_v.at[b * B + i]], sem,
                             add=True)
        return carry

    lax.fori_loop(0, NB, fire, 0)

    def drain(b, carry):
        for i in range(B):
            pltpu.make_async_copy(ones_v, acc_s.at[dst_v.at[b * B + i]],
                                  sem).wait()
        return carry

    lax.fori_loop(0, NB, drain, 0)
    plsc.subcore_barrier()
    pltpu.sync_copy(acc_s.at[pl.ds(rs, RPT)], out_hbm.at[c, pl.ds(rs, RPT)])


@functools.lru_cache(maxsize=None)
def _make_propagate(w):
    nbuf = NB if w <= 4 else 2

    @functools.partial(
        pl.kernel,
        out_type=jax.ShapeDtypeStruct((NC, NP, w), jnp.float32),
        mesh=_sc_mesh(),
        scratch_types=[
            pltpu.VMEM((K, CH), jnp.int32),
            pltpu.VMEM((K, CH), jnp.int32),
            pltpu.VMEM((nbuf, B, CH, w), jnp.float32),
            pltpu.VMEM_SHARED((NP, w), jnp.float32),
            pltpu.VMEM_SHARED((NP, w), jnp.float32),
            pltpu.SemaphoreType.DMA,
            pltpu.SemaphoreType.DMA,
        ],
        compiler_params=pltpu.CompilerParams(use_tc_tiling_on_sc=False),
    )
    def propagate(g_hbm, srcw_hbm, dstw_hbm, out_hbm, src_v, dst_v, rows_v,
                  table_s, acc_s, gsem, ssem):
        c = lax.axis_index("c")
        s = lax.axis_index("s")
        wid = s * NC + c
        pltpu.sync_copy(srcw_hbm.at[wid], src_v)
        pltpu.sync_copy(dstw_hbm.at[wid], dst_v)
        rs = s * RPT
        pltpu.sync_copy(g_hbm.at[pl.ds(rs, RPT)], table_s.at[pl.ds(rs, RPT)])
        pltpu.sync_copy(g_hbm.at[pl.ds(rs, RPT)], acc_s.at[pl.ds(rs, RPT)])
        plsc.subcore_barrier()

        def fire_gathers(b, h):
            for i in range(B):
                pltpu.async_copy(table_s.at[src_v.at[b * B + i]],
                                 rows_v.at[h, i], gsem)

        def drain_gathers(h):
            for i in range(B):
                pltpu.make_async_copy(table_s.at[src_v.at[0]],
                                      rows_v.at[h, i], gsem).wait()

        def fire_scatters(b, h):
            for i in range(B):
                pltpu.async_copy(rows_v.at[h, i],
                                 acc_s.at[dst_v.at[b * B + i]], ssem, add=True)

        def drain_scatters(h):
            for i in range(B):
                pltpu.make_async_copy(rows_v.at[h, i],
                                      acc_s.at[dst_v.at[0]], ssem).wait()

        fire_gathers(0, 0)

        def body(b, carry):
            h = lax.rem(b, nbuf)
            drain_gathers(h)
            @pl.when(b >= nbuf - 1)
            def _():
                drain_scatters(lax.rem(b + 1, nbuf))

            @pl.when(b + 1 < NB)
            def _():
                fire_gathers(b + 1, lax.rem(b + 1, nbuf))

            fire_scatters(b, h)
            return carry

        lax.fori_loop(0, NB, body, 0)
        for bb in range(max(0, NB - nbuf + 1), NB):
            drain_scatters(bb % nbuf)
        plsc.subcore_barrier()
        pltpu.sync_copy(acc_s.at[pl.ds(rs, RPT)], out_hbm.at[c, pl.ds(rs, RPT)])

    return propagate


def _tc_head_body(dacc_ref, x_ref, w1_ref, dinv_ref, g_ref):
    deg = dacc_ref[0] + dacc_ref[1] - 1.0
    dinv = lax.rsqrt(deg)
    dinv_ref[...] = dinv
    g_ref[...] = jnp.dot(x_ref[...], w1_ref[...],
                         preferred_element_type=jnp.float32) * dinv


_tc_head = pl.pallas_call(
    _tc_head_body,
    grid=(NP // RB,),
    in_specs=[
        pl.BlockSpec((NC, RB, 1), lambda i: (0, i, 0)),
        pl.BlockSpec((RB, 128), lambda i: (i, 0)),
        pl.BlockSpec((128, 16), lambda i: (0, 0)),
    ],
    out_specs=[
        pl.BlockSpec((RB, 1), lambda i: (i, 0)),
        pl.BlockSpec((RB, 16), lambda i: (i, 0)),
    ],
    out_shape=[
        jax.ShapeDtypeStruct((NP, 1), jnp.float32),
        jax.ShapeDtypeStruct((NP, 16), jnp.float32),
    ],
)


def _tc_mid_body(acc_ref, g_ref, dinv_ref, wn_ref, b_ref, gn_ref):
    dinv = dinv_ref[...]
    act = jnp.tanh(dinv * (acc_ref[0] + acc_ref[1] - g_ref[...]) + b_ref[...])
    gn_ref[...] = jnp.dot(act, wn_ref[...],
                          preferred_element_type=jnp.float32) * dinv


@functools.lru_cache(maxsize=None)
def _make_tc_mid(w, w2):
    return pl.pallas_call(
        _tc_mid_body,
        grid=(NP // RB,),
        in_specs=[
            pl.BlockSpec((NC, RB, w), lambda i: (0, i, 0)),
            pl.BlockSpec((RB, w), lambda i: (i, 0)),
            pl.BlockSpec((RB, 1), lambda i: (i, 0)),
            pl.BlockSpec((w, w2), lambda i: (0, 0)),
            pl.BlockSpec((1, w), lambda i: (0, 0)),
        ],
        out_specs=pl.BlockSpec((RB, w2), lambda i: (i, 0)),
        out_shape=jax.ShapeDtypeStruct((NP, w2), jnp.float32),
    )


def _tc_tail_body(acc_ref, g_ref, dinv_ref, b5_ref, wl_ref, bl_ref,
                  out_ref, h_ref):
    act = jnp.tanh(dinv_ref[...] * (acc_ref[0] + acc_ref[1] - g_ref[...])
                   + b5_ref[...])
    h_ref[...] = act
    out_ref[...] = jnp.dot(act, wl_ref[...],
                           preferred_element_type=jnp.float32) + bl_ref[...]


_tc_tail = pl.pallas_call(
    _tc_tail_body,
    grid=(NP // RB,),
    in_specs=[
        pl.BlockSpec((NC, RB, 2), lambda i: (0, i, 0)),
        pl.BlockSpec((RB, 2), lambda i: (i, 0)),
        pl.BlockSpec((RB, 1), lambda i: (i, 0)),
        pl.BlockSpec((1, 2), lambda i: (0, 0)),
        pl.BlockSpec((2, 16), lambda i: (0, 0)),
        pl.BlockSpec((1, 16), lambda i: (0, 0)),
    ],
    out_specs=[
        pl.BlockSpec((RB, 16), lambda i: (i, 0)),
        pl.BlockSpec((RB, 2), lambda i: (i, 0)),
    ],
    out_shape=[
        jax.ShapeDtypeStruct((NP, 16), jnp.float32),
        jax.ShapeDtypeStruct((NP, 2), jnp.float32),
    ],
)


def kernel(x, edge_index, W1, b1, W2, b2, W3, b3, W4, b4, W5, b5, Wl, bl):
    src = edge_index[0].astype(jnp.int32)
    dst = edge_index[1].astype(jnp.int32)
    pad = jnp.full((EP - E,), N, jnp.int32)
    srcw = jnp.concatenate([src, pad]).reshape(NW, K, CH)
    dstw = jnp.concatenate([dst, pad]).reshape(NW, K, CH)
    xp = jnp.zeros((NP, 128), jnp.float32).at[:N].set(x)
    ones_col = jnp.ones((NP, 1), jnp.float32)

    dacc = _sc_degree(ones_col, dstw)
    dinv, g = _tc_head(dacc, xp, W1)

    acc = _make_propagate(16)(g, srcw, dstw)
    g = _make_tc_mid(16, 4)(acc, g, dinv, W2, b1[None, :])
    acc = _make_propagate(4)(g, srcw, dstw)
    g = _make_tc_mid(4, 4)(acc, g, dinv, W3, b2[None, :])
    acc = _make_propagate(4)(g, srcw, dstw)
    g = _make_tc_mid(4, 2)(acc, g, dinv, W4, b3[None, :])
    acc = _make_propagate(2)(g, srcw, dstw)
    g = _make_tc_mid(2, 2)(acc, g, dinv, W5, b4[None, :])
    acc = _make_propagate(2)(g, srcw, dstw)
    out, h = _tc_tail(acc, g, dinv, b5[None, :], Wl, bl[None, :])
    return (out[:N], h[:N])


# speedup vs baseline: 56.2294x; 1.2177x over previous
import functools

import jax
import jax.numpy as jnp
from jax import lax
from jax.experimental import pallas as pl
from jax.experimental.pallas import tpu as pltpu
from jax.experimental.pallas import tpu_sc as plsc

N = 10000
E = 320000
NP = 10240
NC, NS = 2, 16
NW = NC * NS
CH = 125
K = 80
B = 16
NB = K // B
RPT = NP // NS
RB = NP


def _sc_mesh():
    return plsc.VectorSubcoreMesh(core_axis_name="c", subcore_axis_name="s")


@functools.partial(
    pl.kernel,
    out_type=jax.ShapeDtypeStruct((NC, NP, 1), jnp.float32),
    mesh=_sc_mesh(),
    scratch_types=[
        pltpu.VMEM((K, CH), jnp.int32),
        pltpu.VMEM((CH, 1), jnp.float32),
        pltpu.VMEM_SHARED((NP, 1), jnp.float32),
        pltpu.SemaphoreType.DMA,
    ],
    compiler_params=pltpu.CompilerParams(use_tc_tiling_on_sc=False),
)
def _sc_degree(ones_hbm, dstw_hbm, out_hbm, dst_v, ones_v, acc_s, sem):
    c = lax.axis_index("c")
    s = lax.axis_index("s")
    wid = s * NC + c
    pltpu.sync_copy(dstw_hbm.at[wid], dst_v)
    pltpu.sync_copy(ones_hbm.at[pl.ds(0, CH)], ones_v)
    rs = s * RPT
    pltpu.sync_copy(ones_hbm.at[pl.ds(rs, RPT)], acc_s.at[pl.ds(rs, RPT)])
    plsc.subcore_barrier()

    def fire(b, carry):
        for i in range(B):
            pltpu.async_copy(ones_v, acc_s.at[dst_v.at[b * B + i]], sem,
                             add=True)
        return carry

    lax.fori_loop(0, NB, fire, 0)

    def drain(b, carry):
        for i in range(B):
            pltpu.make_async_copy(ones_v, acc_s.at[dst_v.at[b * B + i]],
                                  sem).wait()
        return carry

    lax.fori_loop(0, NB, drain, 0)
    plsc.subcore_barrier()
    pltpu.sync_copy(acc_s.at[pl.ds(rs, RPT)], out_hbm.at[c, pl.ds(rs, RPT)])


@functools.lru_cache(maxsize=None)
def _make_propagate(w):
    nbuf = NB if w <= 4 else 2

    @functools.partial(
        pl.kernel,
        out_type=jax.ShapeDtypeStruct((NC, NP, w), jnp.float32),
        mesh=_sc_mesh(),
        scratch_types=[
            pltpu.VMEM((K, CH), jnp.int32),
            pltpu.VMEM((K, CH), jnp.int32),
            pltpu.VMEM((nbuf, B, CH, w), jnp.float32),
            pltpu.VMEM_SHARED((NP, w), jnp.float32),
            pltpu.VMEM_SHARED((NP, w), jnp.float32),
            pltpu.SemaphoreType.DMA,
            pltpu.SemaphoreType.DMA,
        ],
        compiler_params=pltpu.CompilerParams(use_tc_tiling_on_sc=False),
    )
    def propagate(g_hbm, srcw_hbm, dstw_hbm, out_hbm, src_v, dst_v, rows_v,
                  table_s, acc_s, gsem, ssem):
        c = lax.axis_index("c")
        s = lax.axis_index("s")
        wid = s * NC + c
        pltpu.sync_copy(srcw_hbm.at[wid], src_v)
        pltpu.sync_copy(dstw_hbm.at[wid], dst_v)
        rs = s * RPT
        pltpu.sync_copy(g_hbm.at[pl.ds(rs, RPT)], table_s.at[pl.ds(rs, RPT)])
        pltpu.sync_copy(g_hbm.at[pl.ds(rs, RPT)], acc_s.at[pl.ds(rs, RPT)])
        plsc.subcore_barrier()

        def fire_gathers(b, h):
            for i in range(B):
                pltpu.async_copy(table_s.at[src_v.at[b * B + i]],
                                 rows_v.at[h, i], gsem)

        def drain_gathers(h):
            for i in range(B):
                pltpu.make_async_copy(table_s.at[src_v.at[0]],
                                      rows_v.at[h, i], gsem).wait()

        def fire_scatters(b, h):
            for i in range(B):
                pltpu.async_copy(rows_v.at[h, i],
                                 acc_s.at[dst_v.at[b * B + i]], ssem, add=True)

        def drain_scatters(h):
            for i in range(B):
                pltpu.make_async_copy(rows_v.at[h, i],
                                      acc_s.at[dst_v.at[0]], ssem).wait()

        fire_gathers(0, 0)

        def body(b, carry):
            h = lax.rem(b, nbuf)
            drain_gathers(h)
            @pl.when(b >= nbuf - 1)
            def _():
                drain_scatters(lax.rem(b + 1, nbuf))

            @pl.when(b + 1 < NB)
            def _():
                fire_gathers(b + 1, lax.rem(b + 1, nbuf))

            fire_scatters(b, h)
            return carry

        lax.fori_loop(0, NB, body, 0)
        for bb in range(max(0, NB - nbuf + 1), NB):
            drain_scatters(bb % nbuf)
        plsc.subcore_barrier()
        pltpu.sync_copy(acc_s.at[pl.ds(rs, RPT)], out_hbm.at[c, pl.ds(rs, RPT)])

    return propagate


def _tc_head_body(dacc_ref, x_ref, w1_ref, dinv_ref, g_ref):
    deg = dacc_ref[0] + dacc_ref[1] - 1.0
    dinv = lax.rsqrt(deg)
    dinv_ref[...] = dinv
    g_ref[...] = jnp.dot(x_ref[...], w1_ref[...],
                         preferred_element_type=jnp.float32) * dinv


_tc_head = pl.pallas_call(
    _tc_head_body,
    grid=(NP // RB,),
    in_specs=[
        pl.BlockSpec((NC, RB, 1), lambda i: (0, i, 0)),
        pl.BlockSpec((RB, 128), lambda i: (i, 0)),
        pl.BlockSpec((128, 16), lambda i: (0, 0)),
    ],
    out_specs=[
        pl.BlockSpec((RB, 1), lambda i: (i, 0)),
        pl.BlockSpec((RB, 16), lambda i: (i, 0)),
    ],
    out_shape=[
        jax.ShapeDtypeStruct((NP, 1), jnp.float32),
        jax.ShapeDtypeStruct((NP, 16), jnp.float32),
    ],
)


def _tc_mid_body(acc_ref, g_ref, dinv_ref, wn_ref, b_ref, gn_ref):
    dinv = dinv_ref[...]
    act = jnp.tanh(dinv * (acc_ref[0] + acc_ref[1] - g_ref[...]) + b_ref[...])
    gn_ref[...] = jnp.dot(act, wn_ref[...],
                          preferred_element_type=jnp.float32) * dinv


@functools.lru_cache(maxsize=None)
def _make_tc_mid(w, w2):
    return pl.pallas_call(
        _tc_mid_body,
        grid=(NP // RB,),
        in_specs=[
            pl.BlockSpec((NC, RB, w), lambda i: (0, i, 0)),
            pl.BlockSpec((RB, w), lambda i: (i, 0)),
            pl.BlockSpec((RB, 1), lambda i: (i, 0)),
            pl.BlockSpec((w, w2), lambda i: (0, 0)),
            pl.BlockSpec((1, w), lambda i: (0, 0)),
        ],
        out_specs=pl.BlockSpec((RB, w2), lambda i: (i, 0)),
        out_shape=jax.ShapeDtypeStruct((NP, w2), jnp.float32),
    )


def _tc_tail_body(acc_ref, g_ref, dinv_ref, b5_ref, wl_ref, bl_ref,
                  out_ref, h_ref):
    act = jnp.tanh(dinv_ref[...] * (acc_ref[0] + acc_ref[1] - g_ref[...])
                   + b5_ref[...])
    h_ref[...] = act
    out_ref[...] = jnp.dot(act, wl_ref[...],
                           preferred_element_type=jnp.float32) + bl_ref[...]


_tc_tail = pl.pallas_call(
    _tc_tail_body,
    grid=(NP // RB,),
    in_specs=[
        pl.BlockSpec((NC, RB, 2), lambda i: (0, i, 0)),
        pl.BlockSpec((RB, 2), lambda i: (i, 0)),
        pl.BlockSpec((RB, 1), lambda i: (i, 0)),
        pl.BlockSpec((1, 2), lambda i: (0, 0)),
        pl.BlockSpec((2, 16), lambda i: (0, 0)),
        pl.BlockSpec((1, 16), lambda i: (0, 0)),
    ],
    out_specs=[
        pl.BlockSpec((RB, 16), lambda i: (i, 0)),
        pl.BlockSpec((RB, 2), lambda i: (i, 0)),
    ],
    out_shape=[
        jax.ShapeDtypeStruct((NP, 16), jnp.float32),
        jax.ShapeDtypeStruct((NP, 2), jnp.float32),
    ],
)


def kernel(x, edge_index, W1, b1, W2, b2, W3, b3, W4, b4, W5, b5, Wl, bl):
    src = edge_index[0].astype(jnp.int32)
    dst = edge_index[1].astype(jnp.int32)
    srcw = src.reshape(NW, K, CH)
    dstw = dst.reshape(NW, K, CH)
    xp = jnp.zeros((NP, 128), jnp.float32).at[:N].set(x)
    ones_col = jnp.ones((NP, 1), jnp.float32)

    dacc = _sc_degree(ones_col, dstw)
    dinv, g = _tc_head(dacc, xp, W1)

    acc = _make_propagate(16)(g, srcw, dstw)
    g = _make_tc_mid(16, 4)(acc, g, dinv, W2, b1[None, :])
    acc = _make_propagate(4)(g, srcw, dstw)
    g = _make_tc_mid(4, 4)(acc, g, dinv, W3, b2[None, :])
    acc = _make_propagate(4)(g, srcw, dstw)
    g = _make_tc_mid(4, 2)(acc, g, dinv, W4, b3[None, :])
    acc = _make_propagate(2)(g, srcw, dstw)
    g = _make_tc_mid(2, 2)(acc, g, dinv, W5, b4[None, :])
    acc = _make_propagate(2)(g, srcw, dstw)
    out, h = _tc_tail(acc, g, dinv, b5[None, :], Wl, bl[None, :])
    return (out[:N], h[:N])


# speedup vs baseline: 74.9472x; 1.3329x over previous
import functools

import jax
import jax.numpy as jnp
from jax import lax
from jax.experimental import pallas as pl
from jax.experimental.pallas import tpu as pltpu
from jax.experimental.pallas import tpu_sc as plsc

N = 10000
E = 320000
NP = 10240
NC, NS = 2, 16
NW = NC * NS
CH = 125
K = 80
RPT = NP // NS
L = 16


def _sc_mesh():
    return plsc.VectorSubcoreMesh(core_axis_name="c", subcore_axis_name="s")


def _full(v):
    return jnp.full((L,), v, jnp.int32)


@functools.partial(
    pl.kernel,
    out_type=jax.ShapeDtypeStruct((NC, NP, 1), jnp.float32),
    mesh=_sc_mesh(),
    scratch_types=[
        pltpu.VMEM((K, CH), jnp.int32),
        pltpu.VMEM((CH, 1), jnp.float32),
        pltpu.VMEM_SHARED((NP, 1), jnp.float32),
        pltpu.SemaphoreType.DMA,
    ],
    compiler_params=pltpu.CompilerParams(use_tc_tiling_on_sc=False, needs_layout_passes=False),
)
def _sc_degree(ones_hbm, dstw_hbm, out_hbm, dst_v, ones_v, acc_s, sem):
    c = lax.axis_index("c")
    s = lax.axis_index("s")
    wid = s * NC + c
    pltpu.sync_copy(dstw_hbm.at[wid], dst_v)
    pltpu.sync_copy(ones_hbm.at[pl.ds(0, CH)], ones_v)
    rs = s * RPT
    pltpu.sync_copy(ones_hbm.at[pl.ds(rs, RPT)], acc_s.at[pl.ds(rs, RPT)])
    plsc.subcore_barrier()

    def fire(b, carry):
        for i in range(16):
            pltpu.async_copy(ones_v, acc_s.at[dst_v.at[b * 16 + i]], sem,
                             add=True)
        return carry

    lax.fori_loop(0, K // 16, fire, 0)

    def drain(b, carry):
        for i in range(16):
            pltpu.make_async_copy(ones_v, acc_s.at[dst_v.at[b * 16 + i]],
                                  sem).wait()
        return carry

    lax.fori_loop(0, K // 16, drain, 0)
    plsc.subcore_barrier()
    pltpu.sync_copy(acc_s.at[pl.ds(rs, RPT)], out_hbm.at[c, pl.ds(rs, RPT)])


def _propagate_phase(src_v, dst_v, rows_v, table_s, acc_s, gsem, ssem,
                     bsz, nbuf):
    nb = K // bsz

    def fire_gathers(b, h):
        for i in range(bsz):
            pltpu.async_copy(table_s.at[src_v.at[b * bsz + i]],
                             rows_v.at[h, i], gsem)

    def drain_gathers(h):
        for i in range(bsz):
            pltpu.make_async_copy(table_s.at[src_v.at[0]],
                                  rows_v.at[h, i], gsem).wait()

    def fire_scatters(b, h):
        for i in range(bsz):
            pltpu.async_copy(rows_v.at[h, i],
                             acc_s.at[dst_v.at[b * bsz + i]], ssem, add=True)

    def drain_scatters(h):
        for i in range(bsz):
            pltpu.make_async_copy(rows_v.at[h, i],
                                  acc_s.at[dst_v.at[0]], ssem).wait()

    fire_gathers(0, 0)

    def body(b, carry):
        h = lax.rem(b, nbuf)
        drain_gathers(h)
        @pl.when(b >= nbuf - 1)
        def _():
            drain_scatters(lax.rem(b + 1, nbuf))

        @pl.when(b + 1 < nb)
        def _():
            fire_gathers(b + 1, lax.rem(b + 1, nbuf))

        fire_scatters(b, h)
        return carry

    lax.fori_loop(0, nb, body, 0)
    for bb in range(max(0, nb - nbuf + 1), nb):
        drain_scatters(bb % nbuf)


def _tanh(x):
    e2 = jnp.exp(x + x)
    return 1.0 - 2.0 / (e2 + 1.0)


@functools.partial(
    pl.kernel,
    out_type=[
        jax.ShapeDtypeStruct((NC, NP, 16), jnp.float32),
        jax.ShapeDtypeStruct((NC, NP), jnp.float32),
    ],
    mesh=_sc_mesh(),
    scratch_types=[
        pltpu.VMEM((K, CH), jnp.int32),
        pltpu.VMEM((K, CH), jnp.int32),
        pltpu.VMEM((2, 10, CH, 16), jnp.float32),
        pltpu.VMEM((RPT, 16), jnp.float32),
        pltpu.VMEM((RPT, 1), jnp.float32),
        pltpu.VMEM((RPT, 1), jnp.float32),
        pltpu.VMEM((RPT,), jnp.float32),
        pltpu.VMEM((RPT, 16), jnp.float32),
        pltpu.VMEM((RPT, 16), jnp.float32),
        pltpu.VMEM_SHARED((NP, 16), jnp.float32),
        pltpu.VMEM_SHARED((NP, 16), jnp.float32),
        pltpu.SemaphoreType.DMA,
        pltpu.SemaphoreType.DMA,
    ],
    compiler_params=pltpu.CompilerParams(use_tc_tiling_on_sc=False, needs_layout_passes=False),
)
def _sc_layer1(h1_hbm, dacc_hbm, srcw_hbm, dstw_hbm, out_hbm, dinv_hbm,
               src_v, dst_v, rows_v, h_v, d0_v, d1_v, dinv_v, g_v, gh_v,
               table_s, acc_s, gsem, ssem):
    c = lax.axis_index("c")
    s = lax.axis_index("s")
    wid = s * NC + c
    pltpu.sync_copy(srcw_hbm.at[wid], src_v)
    pltpu.sync_copy(dstw_hbm.at[wid], dst_v)
    rs = s * RPT
    pltpu.sync_copy(h1_hbm.at[pl.ds(rs, RPT)], h_v)
    pltpu.sync_copy(dacc_hbm.at[0, pl.ds(rs, RPT)], d0_v)
    pltpu.sync_copy(dacc_hbm.at[1, pl.ds(rs, RPT)], d1_v)

    z16 = _full(0)

    def chunk(kk, carry):
        rows = lax.iota(jnp.int32, L) + kk * L
        deg = (plsc.load_gather(d0_v, [rows, z16])
               + plsc.load_gather(d1_v, [rows, z16])) - 1.0
        i = plsc.bitcast(deg, jnp.int32)
        y = plsc.bitcast(jnp.int32(0x5F3759DF) - (i >> 1), jnp.float32)
        for _ in range(4):
            y = y * (1.5 - 0.5 * deg * y * y)
        plsc.store_scatter(dinv_v, [rows], y)
        for f in range(16):
            gcol = plsc.load_gather(h_v, [rows, _full(f)]) * y
            plsc.store_scatter(g_v, [rows, _full(f)], gcol)
            plsc.store_scatter(gh_v, [rows, _full(f)], gcol * 0.5)
        return carry

    lax.fori_loop(0, RPT // L, chunk, 0)
    pltpu.sync_copy(g_v, table_s.at[pl.ds(rs, RPT)])
    pltpu.sync_copy(gh_v, acc_s.at[pl.ds(rs, RPT)])
    pltpu.sync_copy(dinv_v, dinv_hbm.at[c, pl.ds(rs, RPT)])
    plsc.subcore_barrier()
    _propagate_phase(src_v, dst_v, rows_v, table_s, acc_s, gsem, ssem,
                     bsz=10, nbuf=2)
    plsc.subcore_barrier()
    pltpu.sync_copy(acc_s.at[pl.ds(rs, RPT)], out_hbm.at[c, pl.ds(rs, RPT)])


@functools.lru_cache(maxsize=None)
def _make_sc_layer(w_in, w_out):
    bsz = 16
    nbuf = 2

    @functools.partial(
        pl.kernel,
        out_type=jax.ShapeDtypeStruct((NC, NP, w_out), jnp.float32),
        mesh=_sc_mesh(),
        scratch_types=[
            pltpu.VMEM((K, CH), jnp.int32),
            pltpu.VMEM((K, CH), jnp.int32),
            pltpu.VMEM((nbuf, bsz, CH, w_out), jnp.float32),
            pltpu.VMEM((RPT, w_in), jnp.float32),
            pltpu.VMEM((RPT, w_in), jnp.float32),
            pltpu.VMEM((RPT,), jnp.float32),
            pltpu.VMEM((w_in, w_out), jnp.float32),
            pltpu.VMEM((w_in,), jnp.float32),
            pltpu.VMEM((RPT, w_out), jnp.float32),
            pltpu.VMEM((RPT, w_out), jnp.float32),
            pltpu.VMEM_SHARED((NP, w_out), jnp.float32),
            pltpu.VMEM_SHARED((NP, w_out), jnp.float32),
            pltpu.SemaphoreType.DMA,
            pltpu.SemaphoreType.DMA,
        ],
        compiler_params=pltpu.CompilerParams(use_tc_tiling_on_sc=False, needs_layout_passes=False),
    )
    def sc_layer(accp_hbm, dinv2_hbm, w_hbm, b_hbm, srcw_hbm, dstw_hbm,
                 out_hbm, src_v, dst_v, rows_v, a0_v, a1_v, dinv_v, w_v, b_v,
                 g_v, gh_v, table_s, acc_s, gsem, ssem):
        c = lax.axis_index("c")
        s = lax.axis_index("s")
        wid = s * NC + c
        pltpu.sync_copy(srcw_hbm.at[wid], src_v)
        pltpu.sync_copy(dstw_hbm.at[wid], dst_v)
        rs = s * RPT
        pltpu.sync_copy(accp_hbm.at[0, pl.ds(rs, RPT)], a0_v)
        pltpu.sync_copy(accp_hbm.at[1, pl.ds(rs, RPT)], a1_v)
        pltpu.sync_copy(dinv2_hbm.at[0, pl.ds(rs, RPT)], dinv_v)
        pltpu.sync_copy(w_hbm, w_v)
        pltpu.sync_copy(b_hbm, b_v)

        wsc = [[plsc.load_gather(w_v, [_full(f), _full(j)])
                for j in range(w_out)] for f in range(w_in)]
        bsc = [plsc.load_gather(b_v, [_full(f)]) for f in range(w_in)]

        def chunk(kk, carry):
            rows = lax.iota(jnp.int32, L) + kk * L
            d16 = plsc.load_gather(dinv_v, [rows])
            acts = []
            for f in range(w_in):
                pre = (plsc.load_gather(a0_v, [rows, _full(f)])
                       + plsc.load_gather(a1_v, [rows, _full(f)])) * d16 \
                      + bsc[f]
                acts.append(_tanh(pre))
            for j in range(w_out):
                acc = acts[0] * wsc[0][j]
                for f in range(1, w_in):
                    acc = acc + acts[f] * wsc[f][j]
                gcol = acc * d16
                plsc.store_scatter(g_v, [rows, _full(j)], gcol)
                plsc.store_scatter(gh_v, [rows, _full(j)], gcol * 0.5)
            return carry

        lax.fori_loop(0, RPT // L, chunk, 0)
        pltpu.sync_copy(g_v, table_s.at[pl.ds(rs, RPT)])
        pltpu.sync_copy(gh_v, acc_s.at[pl.ds(rs, RPT)])
        plsc.subcore_barrier()
        _propagate_phase(src_v, dst_v, rows_v, table_s, acc_s, gsem, ssem,
                         bsz=bsz, nbuf=nbuf)
        plsc.subcore_barrier()
        pltpu.sync_copy(acc_s.at[pl.ds(rs, RPT)], out_hbm.at[c, pl.ds(rs, RPT)])

    return sc_layer


def _tc_head_body(x_ref, w1_ref, h_ref):
    h_ref[...] = jnp.dot(x_ref[...], w1_ref[...],
                         preferred_element_type=jnp.float32)


_tc_head = pl.pallas_call(
    _tc_head_body,
    out_shape=jax.ShapeDtypeStruct((NP, 16), jnp.float32),
)


def _tc_tail_body(acc_ref, dinv_ref, b5_ref, wl_ref, bl_ref, out_ref, h_ref):
    dinv = dinv_ref[0].reshape(NP, 1)
    act = jnp.tanh(dinv * (acc_ref[0] + acc_ref[1]) + b5_ref[...])
    h_ref[...] = act[:N]
    out_ref[...] = jnp.dot(act[:N], wl_ref[...],
                           preferred_element_type=jnp.float32) + bl_ref[...]


_tc_tail = pl.pallas_call(
    _tc_tail_body,
    out_shape=[
        jax.ShapeDtypeStruct((N, 16), jnp.float32),
        jax.ShapeDtypeStruct((N, 2), jnp.float32),
    ],
)


def kernel(x, edge_index, W1, b1, W2, b2, W3, b3, W4, b4, W5, b5, Wl, bl):
    src = edge_index[0].astype(jnp.int32)
    dst = edge_index[1].astype(jnp.int32)
    srcw = src.reshape(NW, K, CH)
    dstw = dst.reshape(NW, K, CH)
    xp = jnp.zeros((NP, 128), jnp.float32).at[:N].set(x)
    ones_col = jnp.ones((NP, 1), jnp.float32)

    dacc = _sc_degree(ones_col, dstw)
    h1 = _tc_head(xp, W1)
    acc, dinv2 = _sc_layer1(h1, dacc, srcw, dstw)
    acc = _make_sc_layer(16, 4)(acc, dinv2, W2, b1, srcw, dstw)
    acc = _make_sc_layer(4, 4)(acc, dinv2, W3, b2, srcw, dstw)
    acc = _make_sc_layer(4, 2)(acc, dinv2, W4, b3, srcw, dstw)
    acc = _make_sc_layer(2, 2)(acc, dinv2, W5, b4, srcw, dstw)
    out, h = _tc_tail(acc, dinv2, b5[None, :], Wl, bl[None, :])
    return (out, h)


# speedup vs baseline: 81.2926x; 1.0847x over previous
import functools

import jax
import jax.numpy as jnp
from jax import lax
from jax.experimental import pallas as pl
from jax.experimental.pallas import tpu as pltpu
from jax.experimental.pallas import tpu_sc as plsc

N = 10000
E = 320000
NP = 10240
NC, NS = 2, 16
NW = NC * NS
CH = 125
K = 80
RPT = NP // NS
L = 16


def _sc_mesh():
    return plsc.VectorSubcoreMesh(core_axis_name="c", subcore_axis_name="s")


def _full(v):
    return jnp.full((L,), v, jnp.int32)


@functools.partial(
    pl.kernel,
    out_type=jax.ShapeDtypeStruct((NC, NP, 1), jnp.float32),
    mesh=_sc_mesh(),
    scratch_types=[
        pltpu.VMEM((K, CH), jnp.int32),
        pltpu.VMEM((CH, 1), jnp.float32),
        pltpu.VMEM_SHARED((NP, 1), jnp.float32),
        pltpu.SemaphoreType.DMA,
    ],
    compiler_params=pltpu.CompilerParams(use_tc_tiling_on_sc=False, needs_layout_passes=False),
)
def _sc_degree(ones_hbm, dstw_hbm, out_hbm, dst_v, ones_v, acc_s, sem):
    c = lax.axis_index("c")
    s = lax.axis_index("s")
    wid = s * NC + c
    pltpu.sync_copy(dstw_hbm.at[wid], dst_v)
    pltpu.sync_copy(ones_hbm.at[pl.ds(0, CH)], ones_v)
    rs = s * RPT
    pltpu.sync_copy(ones_hbm.at[pl.ds(rs, RPT)], acc_s.at[pl.ds(rs, RPT)])
    plsc.subcore_barrier()

    def fire(b, carry):
        for i in range(16):
            pltpu.async_copy(ones_v, acc_s.at[dst_v.at[b * 16 + i]], sem,
                             add=True)
        return carry

    lax.fori_loop(0, K // 16, fire, 0)

    def drain(b, carry):
        for i in range(16):
            pltpu.make_async_copy(ones_v, acc_s.at[dst_v.at[b * 16 + i]],
                                  sem).wait()
        return carry

    lax.fori_loop(0, K // 16, drain, 0)
    plsc.subcore_barrier()
    pltpu.sync_copy(acc_s.at[pl.ds(rs, RPT)], out_hbm.at[c, pl.ds(rs, RPT)])


def _propagate_phase(src_v, dst_v, rows_v, table_s, acc_s, gsem, ssem,
                     bsz, nbuf):
    nb = K // bsz

    def fire_gathers(b, h):
        for i in range(bsz):
            pltpu.async_copy(table_s.at[src_v.at[b * bsz + i]],
                             rows_v.at[h, i], gsem)

    def drain_gathers(h):
        for i in range(bsz):
            pltpu.make_async_copy(table_s.at[src_v.at[0]],
                                  rows_v.at[h, i], gsem).wait()

    def fire_scatters(b, h):
        for i in range(bsz):
            pltpu.async_copy(rows_v.at[h, i],
                             acc_s.at[dst_v.at[b * bsz + i]], ssem, add=True)

    def drain_scatters(h):
        for i in range(bsz):
            pltpu.make_async_copy(rows_v.at[h, i],
                                  acc_s.at[dst_v.at[0]], ssem).wait()

    fire_gathers(0, 0)

    def body(b, carry):
        h = lax.rem(b, nbuf)
        drain_gathers(h)
        @pl.when(b >= nbuf - 1)
        def _():
            drain_scatters(lax.rem(b + 1, nbuf))

        @pl.when(b + 1 < nb)
        def _():
            fire_gathers(b + 1, lax.rem(b + 1, nbuf))

        fire_scatters(b, h)
        return carry

    lax.fori_loop(0, nb, body, 0)
    for bb in range(max(0, nb - nbuf + 1), nb):
        drain_scatters(bb % nbuf)


def _tanh(x):
    e2 = jnp.exp(x + x)
    return 1.0 - 2.0 / (e2 + 1.0)


@functools.partial(
    pl.kernel,
    out_type=[
        jax.ShapeDtypeStruct((NC, NP, 16), jnp.float32),
        jax.ShapeDtypeStruct((NC, NP), jnp.float32),
    ],
    mesh=_sc_mesh(),
    scratch_types=[
        pltpu.VMEM((K, CH), jnp.int32),
        pltpu.VMEM((K, CH), jnp.int32),
        pltpu.VMEM((2, 10, CH, 16), jnp.float32),
        pltpu.VMEM((RPT, 16), jnp.float32),
        pltpu.VMEM((RPT, 1), jnp.float32),
        pltpu.VMEM((RPT, 1), jnp.float32),
        pltpu.VMEM((RPT,), jnp.float32),
        pltpu.VMEM((RPT, 16), jnp.float32),
        pltpu.VMEM((RPT, 16), jnp.float32),
        pltpu.VMEM_SHARED((NP, 16), jnp.float32),
        pltpu.VMEM_SHARED((NP, 16), jnp.float32),
        pltpu.SemaphoreType.DMA,
        pltpu.SemaphoreType.DMA,
    ],
    compiler_params=pltpu.CompilerParams(use_tc_tiling_on_sc=False, needs_layout_passes=False),
)
def _sc_layer1(h1_hbm, dacc_hbm, srcw_hbm, dstw_hbm, out_hbm, dinv_hbm,
               src_v, dst_v, rows_v, h_v, d0_v, d1_v, dinv_v, g_v, gh_v,
               table_s, acc_s, gsem, ssem):
    c = lax.axis_index("c")
    s = lax.axis_index("s")
    wid = s * NC + c
    rs = s * RPT
    i1 = pltpu.async_copy(srcw_hbm.at[wid], src_v, ssem)
    i2 = pltpu.async_copy(dstw_hbm.at[wid], dst_v, ssem)
    p1 = pltpu.async_copy(h1_hbm.at[pl.ds(rs, RPT)], h_v, gsem)
    p2 = pltpu.async_copy(dacc_hbm.at[0, pl.ds(rs, RPT)], d0_v, gsem)
    p3 = pltpu.async_copy(dacc_hbm.at[1, pl.ds(rs, RPT)], d1_v, gsem)
    p1.wait()
    p2.wait()
    p3.wait()

    z16 = _full(0)

    def chunk(kk, carry):
        rows = lax.iota(jnp.int32, L) + kk * L
        deg = (plsc.load_gather(d0_v, [rows, z16])
               + plsc.load_gather(d1_v, [rows, z16])) - 1.0
        i = plsc.bitcast(deg, jnp.int32)
        y = plsc.bitcast(jnp.int32(0x5F3759DF) - (i >> 1), jnp.float32)
        for _ in range(4):
            y = y * (1.5 - 0.5 * deg * y * y)
        plsc.store_scatter(dinv_v, [rows], y)
        for f in range(16):
            gcol = plsc.load_gather(h_v, [rows, _full(f)]) * y
            plsc.store_scatter(g_v, [rows, _full(f)], gcol)
            plsc.store_scatter(gh_v, [rows, _full(f)], gcol * 0.5)
        return carry

    lax.fori_loop(0, RPT // L, chunk, 0)
    pltpu.sync_copy(g_v, table_s.at[pl.ds(rs, RPT)])
    pltpu.sync_copy(gh_v, acc_s.at[pl.ds(rs, RPT)])
    pltpu.sync_copy(dinv_v, dinv_hbm.at[c, pl.ds(rs, RPT)])
    i1.wait()
    i2.wait()
    plsc.subcore_barrier()
    _propagate_phase(src_v, dst_v, rows_v, table_s, acc_s, gsem, ssem,
                     bsz=10, nbuf=2)
    plsc.subcore_barrier()
    pltpu.sync_copy(acc_s.at[pl.ds(rs, RPT)], out_hbm.at[c, pl.ds(rs, RPT)])


@functools.lru_cache(maxsize=None)
def _make_sc_layer(w_in, w_out):
    bsz = 16
    nbuf = 2

    @functools.partial(
        pl.kernel,
        out_type=jax.ShapeDtypeStruct((NC, NP, w_out), jnp.float32),
        mesh=_sc_mesh(),
        scratch_types=[
            pltpu.VMEM((K, CH), jnp.int32),
            pltpu.VMEM((K, CH), jnp.int32),
            pltpu.VMEM((nbuf, bsz, CH, w_out), jnp.float32),
            pltpu.VMEM((RPT, w_in), jnp.float32),
            pltpu.VMEM((RPT, w_in), jnp.float32),
            pltpu.VMEM((RPT,), jnp.float32),
            pltpu.VMEM((w_in, w_out), jnp.float32),
            pltpu.VMEM((w_in,), jnp.float32),
            pltpu.VMEM((RPT, w_out), jnp.float32),
            pltpu.VMEM((RPT, w_out), jnp.float32),
            pltpu.VMEM_SHARED((NP, w_out), jnp.float32),
            pltpu.VMEM_SHARED((NP, w_out), jnp.float32),
            pltpu.SemaphoreType.DMA,
            pltpu.SemaphoreType.DMA,
        ],
        compiler_params=pltpu.CompilerParams(use_tc_tiling_on_sc=False, needs_layout_passes=False),
    )
    def sc_layer(accp_hbm, dinv2_hbm, w_hbm, b_hbm, srcw_hbm, dstw_hbm,
                 out_hbm, src_v, dst_v, rows_v, a0_v, a1_v, dinv_v, w_v, b_v,
                 g_v, gh_v, table_s, acc_s, gsem, ssem):
        c = lax.axis_index("c")
        s = lax.axis_index("s")
        wid = s * NC + c
        rs = s * RPT
        i1 = pltpu.async_copy(srcw_hbm.at[wid], src_v, ssem)
        i2 = pltpu.async_copy(dstw_hbm.at[wid], dst_v, ssem)
        ps = [pltpu.async_copy(accp_hbm.at[0, pl.ds(rs, RPT)], a0_v, gsem),
              pltpu.async_copy(accp_hbm.at[1, pl.ds(rs, RPT)], a1_v, gsem),
              pltpu.async_copy(dinv2_hbm.at[0, pl.ds(rs, RPT)], dinv_v, gsem),
              pltpu.async_copy(w_hbm, w_v, gsem),
              pltpu.async_copy(b_hbm, b_v, gsem)]
        for p in ps:
            p.wait()

        wsc = [[plsc.load_gather(w_v, [_full(f), _full(j)])
                for j in range(w_out)] for f in range(w_in)]
        bsc = [plsc.load_gather(b_v, [_full(f)]) for f in range(w_in)]

        def chunk(kk, carry):
            rows = lax.iota(jnp.int32, L) + kk * L
            d16 = plsc.load_gather(dinv_v, [rows])
            acts = []
            for f in range(w_in):
                pre = (plsc.load_gather(a0_v, [rows, _full(f)])
                       + plsc.load_gather(a1_v, [rows, _full(f)])) * d16 \
                      + bsc[f]
                acts.append(_tanh(pre))
            for j in range(w_out):
                acc = acts[0] * wsc[0][j]
                for f in range(1, w_in):
                    acc = acc + acts[f] * wsc[f][j]
                gcol = acc * d16
                plsc.store_scatter(g_v, [rows, _full(j)], gcol)
                plsc.store_scatter(gh_v, [rows, _full(j)], gcol * 0.5)
            return carry

        lax.fori_loop(0, RPT // L, chunk, 0)
        pltpu.sync_copy(g_v, table_s.at[pl.ds(rs, RPT)])
        pltpu.sync_copy(gh_v, acc_s.at[pl.ds(rs, RPT)])
        i1.wait()
        i2.wait()
        plsc.subcore_barrier()
        _propagate_phase(src_v, dst_v, rows_v, table_s, acc_s, gsem, ssem,
                         bsz=bsz, nbuf=nbuf)
        plsc.subcore_barrier()
        pltpu.sync_copy(acc_s.at[pl.ds(rs, RPT)], out_hbm.at[c, pl.ds(rs, RPT)])

    return sc_layer


def _tc_head_body(x_ref, w1_ref, h_ref):
    h_ref[...] = jnp.dot(x_ref[...], w1_ref[...],
                         preferred_element_type=jnp.float32)


_tc_head = pl.pallas_call(
    _tc_head_body,
    out_shape=jax.ShapeDtypeStruct((NP, 16), jnp.float32),
)


def _tc_tail_body(acc_ref, dinv_ref, b5_ref, wl_ref, bl_ref, out_ref, h_ref):
    dinv = dinv_ref[0].reshape(NP, 1)
    act = jnp.tanh(dinv * (acc_ref[0] + acc_ref[1]) + b5_ref[...])
    h_ref[...] = act[:N]
    out_ref[...] = jnp.dot(act[:N], wl_ref[...],
                           preferred_element_type=jnp.float32) + bl_ref[...]


_tc_tail = pl.pallas_call(
    _tc_tail_body,
    out_shape=[
        jax.ShapeDtypeStruct((N, 16), jnp.float32),
        jax.ShapeDtypeStruct((N, 2), jnp.float32),
    ],
)


def kernel(x, edge_index, W1, b1, W2, b2, W3, b3, W4, b4, W5, b5, Wl, bl):
    src = edge_index[0].astype(jnp.int32)
    dst = edge_index[1].astype(jnp.int32)
    srcw = src.reshape(NW, K, CH)
    dstw = dst.reshape(NW, K, CH)
    xp = jnp.zeros((NP, 128), jnp.float32).at[:N].set(x)
    ones_col = jnp.ones((NP, 1), jnp.float32)

    dacc = _sc_degree(ones_col, dstw)
    h1 = _tc_head(xp, W1)
    acc, dinv2 = _sc_layer1(h1, dacc, srcw, dstw)
    acc = _make_sc_layer(16, 4)(acc, dinv2, W2, b1, srcw, dstw)
    acc = _make_sc_layer(4, 4)(acc, dinv2, W3, b2, srcw, dstw)
    acc = _make_sc_layer(4, 2)(acc, dinv2, W4, b3, srcw, dstw)
    acc = _make_sc_layer(2, 2)(acc, dinv2, W5, b4, srcw, dstw)
    out, h = _tc_tail(acc, dinv2, b5[None, :], Wl, bl[None, :])
    return (out, h)


# speedup vs baseline: 86.2830x; 1.0614x over previous
import functools

import jax
import jax.numpy as jnp
from jax import lax
from jax.experimental import pallas as pl
from jax.experimental.pallas import tpu as pltpu
from jax.experimental.pallas import tpu_sc as plsc

N = 10000
E = 320000
NP = 10240
NC, NS = 2, 16
NW = NC * NS
CH = 125
K = 80
RPT = NP // NS
L = 16


def _sc_mesh():
    return plsc.VectorSubcoreMesh(core_axis_name="c", subcore_axis_name="s")


def _full(v):
    return jnp.full((L,), v, jnp.int32)


@functools.partial(
    pl.kernel,
    out_type=jax.ShapeDtypeStruct((NC, NP, 1), jnp.float32),
    mesh=_sc_mesh(),
    scratch_types=[
        pltpu.VMEM((K, CH), jnp.int32),
        pltpu.VMEM((CH, 1), jnp.float32),
        pltpu.VMEM_SHARED((NP, 1), jnp.float32),
        pltpu.SemaphoreType.DMA,
    ],
    compiler_params=pltpu.CompilerParams(use_tc_tiling_on_sc=False, needs_layout_passes=False),
)
def _sc_degree(ones_hbm, eiw_hbm, out_hbm, dst_v, ones_v, acc_s, sem):
    c = lax.axis_index("c")
    s = lax.axis_index("s")
    wid = s * NC + c
    pltpu.sync_copy(eiw_hbm.at[1, wid], dst_v)
    pltpu.sync_copy(ones_hbm.at[pl.ds(0, CH)], ones_v)
    rs = s * RPT
    pltpu.sync_copy(ones_hbm.at[pl.ds(rs, RPT)], acc_s.at[pl.ds(rs, RPT)])
    plsc.subcore_barrier()

    def fire(b, carry):
        for i in range(16):
            pltpu.async_copy(ones_v, acc_s.at[dst_v.at[b * 16 + i]], sem,
                             add=True)
        return carry

    lax.fori_loop(0, K // 16, fire, 0)

    def drain(b, carry):
        for i in range(16):
            pltpu.make_async_copy(ones_v, acc_s.at[dst_v.at[b * 16 + i]],
                                  sem).wait()
        return carry

    lax.fori_loop(0, K // 16, drain, 0)
    plsc.subcore_barrier()
    pltpu.sync_copy(acc_s.at[pl.ds(rs, RPT)], out_hbm.at[c, pl.ds(rs, RPT)])


def _propagate_phase(src_v, dst_v, rows_v, table_s, acc_s, gsem, ssem,
                     bsz, nbuf):
    nb = K // bsz

    def fire_gathers(b, h):
        for i in range(bsz):
            pltpu.async_copy(table_s.at[src_v.at[b * bsz + i]],
                             rows_v.at[h, i], gsem)

    def drain_gathers(h):
        for i in range(bsz):
            pltpu.make_async_copy(table_s.at[src_v.at[0]],
                                  rows_v.at[h, i], gsem).wait()

    def fire_scatters(b, h):
        for i in range(bsz):
            pltpu.async_copy(rows_v.at[h, i],
                             acc_s.at[dst_v.at[b * bsz + i]], ssem, add=True)

    def drain_scatters(h):
        for i in range(bsz):
            pltpu.make_async_copy(rows_v.at[h, i],
                                  acc_s.at[dst_v.at[0]], ssem).wait()

    fire_gathers(0, 0)

    def body(b, carry):
        h = lax.rem(b, nbuf)
        drain_gathers(h)
        @pl.when(b >= nbuf - 1)
        def _():
            drain_scatters(lax.rem(b + 1, nbuf))

        @pl.when(b + 1 < nb)
        def _():
            fire_gathers(b + 1, lax.rem(b + 1, nbuf))

        fire_scatters(b, h)
        return carry

    lax.fori_loop(0, nb, body, 0)
    for bb in range(max(0, nb - nbuf + 1), nb):
        drain_scatters(bb % nbuf)


def _tanh(x):
    e2 = jnp.exp(x + x)
    return 1.0 - 2.0 / (e2 + 1.0)


@functools.partial(
    pl.kernel,
    out_type=[
        jax.ShapeDtypeStruct((NC, NP, 16), jnp.float32),
        jax.ShapeDtypeStruct((NC, NP), jnp.float32),
    ],
    mesh=_sc_mesh(),
    scratch_types=[
        pltpu.VMEM((K, CH), jnp.int32),
        pltpu.VMEM((K, CH), jnp.int32),
        pltpu.VMEM((2, 10, CH, 16), jnp.float32),
        pltpu.VMEM((RPT, 16), jnp.float32),
        pltpu.VMEM((RPT, 1), jnp.float32),
        pltpu.VMEM((RPT, 1), jnp.float32),
        pltpu.VMEM((RPT,), jnp.float32),
        pltpu.VMEM((RPT, 16), jnp.float32),
        pltpu.VMEM((RPT, 16), jnp.float32),
        pltpu.VMEM_SHARED((NP, 16), jnp.float32),
        pltpu.VMEM_SHARED((NP, 16), jnp.float32),
        pltpu.SemaphoreType.DMA,
        pltpu.SemaphoreType.DMA,
    ],
    compiler_params=pltpu.CompilerParams(use_tc_tiling_on_sc=False, needs_layout_passes=False),
)
def _sc_layer1(h1_hbm, dacc_hbm, eiw_hbm, out_hbm, dinv_hbm,
               src_v, dst_v, rows_v, h_v, d0_v, d1_v, dinv_v, g_v, gh_v,
               table_s, acc_s, gsem, ssem):
    c = lax.axis_index("c")
    s = lax.axis_index("s")
    wid = s * NC + c
    rs = s * RPT
    i1 = pltpu.async_copy(eiw_hbm.at[0, wid], src_v, ssem)
    i2 = pltpu.async_copy(eiw_hbm.at[1, wid], dst_v, ssem)
    p1 = pltpu.async_copy(h1_hbm.at[pl.ds(rs, RPT)], h_v, gsem)
    p2 = pltpu.async_copy(dacc_hbm.at[0, pl.ds(rs, RPT)], d0_v, gsem)
    p3 = pltpu.async_copy(dacc_hbm.at[1, pl.ds(rs, RPT)], d1_v, gsem)
    p1.wait()
    p2.wait()
    p3.wait()

    z16 = _full(0)

    def chunk(kk, carry):
        rows = lax.iota(jnp.int32, L) + kk * L
        deg = (plsc.load_gather(d0_v, [rows, z16])
               + plsc.load_gather(d1_v, [rows, z16])) - 1.0
        i = plsc.bitcast(deg, jnp.int32)
        y = plsc.bitcast(jnp.int32(0x5F3759DF) - (i >> 1), jnp.float32)
        for _ in range(4):
            y = y * (1.5 - 0.5 * deg * y * y)
        plsc.store_scatter(dinv_v, [rows], y)
        for f in range(16):
            gcol = plsc.load_gather(h_v, [rows, _full(f)]) * y
            plsc.store_scatter(g_v, [rows, _full(f)], gcol)
            plsc.store_scatter(gh_v, [rows, _full(f)], gcol * 0.5)
        return carry

    lax.fori_loop(0, RPT // L, chunk, 0)
    pltpu.sync_copy(g_v, table_s.at[pl.ds(rs, RPT)])
    pltpu.sync_copy(gh_v, acc_s.at[pl.ds(rs, RPT)])
    pltpu.sync_copy(dinv_v, dinv_hbm.at[c, pl.ds(rs, RPT)])
    i1.wait()
    i2.wait()
    plsc.subcore_barrier()
    _propagate_phase(src_v, dst_v, rows_v, table_s, acc_s, gsem, ssem,
                     bsz=10, nbuf=2)
    plsc.subcore_barrier()
    pltpu.sync_copy(acc_s.at[pl.ds(rs, RPT)], out_hbm.at[c, pl.ds(rs, RPT)])


@functools.lru_cache(maxsize=None)
def _make_sc_layer(w_in, w_out):
    bsz = 16
    nbuf = 2

    @functools.partial(
        pl.kernel,
        out_type=jax.ShapeDtypeStruct((NC, NP, w_out), jnp.float32),
        mesh=_sc_mesh(),
        scratch_types=[
            pltpu.VMEM((K, CH), jnp.int32),
            pltpu.VMEM((K, CH), jnp.int32),
            pltpu.VMEM((nbuf, bsz, CH, w_out), jnp.float32),
            pltpu.VMEM((RPT, w_in), jnp.float32),
            pltpu.VMEM((RPT, w_in), jnp.float32),
            pltpu.VMEM((RPT,), jnp.float32),
            pltpu.VMEM((w_in, w_out), jnp.float32),
            pltpu.VMEM((w_in,), jnp.float32),
            pltpu.VMEM((RPT, w_out), jnp.float32),
            pltpu.VMEM((RPT, w_out), jnp.float32),
            pltpu.VMEM_SHARED((NP, w_out), jnp.float32),
            pltpu.VMEM_SHARED((NP, w_out), jnp.float32),
            pltpu.SemaphoreType.DMA,
            pltpu.SemaphoreType.DMA,
        ],
        compiler_params=pltpu.CompilerParams(use_tc_tiling_on_sc=False, needs_layout_passes=False),
    )
    def sc_layer(accp_hbm, dinv2_hbm, w_hbm, b_hbm, eiw_hbm,
                 out_hbm, src_v, dst_v, rows_v, a0_v, a1_v, dinv_v, w_v, b_v,
                 g_v, gh_v, table_s, acc_s, gsem, ssem):
        c = lax.axis_index("c")
        s = lax.axis_index("s")
        wid = s * NC + c
        rs = s * RPT
        i1 = pltpu.async_copy(eiw_hbm.at[0, wid], src_v, ssem)
        i2 = pltpu.async_copy(eiw_hbm.at[1, wid], dst_v, ssem)
        ps = [pltpu.async_copy(accp_hbm.at[0, pl.ds(rs, RPT)], a0_v, gsem),
              pltpu.async_copy(accp_hbm.at[1, pl.ds(rs, RPT)], a1_v, gsem),
              pltpu.async_copy(dinv2_hbm.at[0, pl.ds(rs, RPT)], dinv_v, gsem),
              pltpu.async_copy(w_hbm, w_v, gsem),
              pltpu.async_copy(b_hbm, b_v, gsem)]
        for p in ps:
            p.wait()

        wsc = [[plsc.load_gather(w_v, [_full(f), _full(j)])
                for j in range(w_out)] for f in range(w_in)]
        bsc = [plsc.load_gather(b_v, [_full(f)]) for f in range(w_in)]

        def chunk(kk, carry):
            rows = lax.iota(jnp.int32, L) + kk * L
            d16 = plsc.load_gather(dinv_v, [rows])
            acts = []
            for f in range(w_in):
                pre = (plsc.load_gather(a0_v, [rows, _full(f)])
                       + plsc.load_gather(a1_v, [rows, _full(f)])) * d16 \
                      + bsc[f]
                acts.append(_tanh(pre))
            for j in range(w_out):
                acc = acts[0] * wsc[0][j]
                for f in range(1, w_in):
                    acc = acc + acts[f] * wsc[f][j]
                gcol = acc * d16
                plsc.store_scatter(g_v, [rows, _full(j)], gcol)
                plsc.store_scatter(gh_v, [rows, _full(j)], gcol * 0.5)
            return carry

        lax.fori_loop(0, RPT // L, chunk, 0)
        pltpu.sync_copy(g_v, table_s.at[pl.ds(rs, RPT)])
        pltpu.sync_copy(gh_v, acc_s.at[pl.ds(rs, RPT)])
        i1.wait()
        i2.wait()
        plsc.subcore_barrier()
        _propagate_phase(src_v, dst_v, rows_v, table_s, acc_s, gsem, ssem,
                         bsz=bsz, nbuf=nbuf)
        plsc.subcore_barrier()
        pltpu.sync_copy(acc_s.at[pl.ds(rs, RPT)], out_hbm.at[c, pl.ds(rs, RPT)])

    return sc_layer


def _tc_head_body(x_ref, w1_ref, h_ref):
    h_ref[...] = jnp.dot(x_ref[...], w1_ref[...],
                         preferred_element_type=jnp.float32)


_tc_head = pl.pallas_call(
    _tc_head_body,
    grid=(10,),
    in_specs=[
        pl.BlockSpec((1024, 128), lambda i: (i, 0)),
        pl.BlockSpec((128, 16), lambda i: (0, 0)),
    ],
    out_specs=pl.BlockSpec((1024, 16), lambda i: (i, 0)),
    out_shape=jax.ShapeDtypeStruct((NP, 16), jnp.float32),
)


def _tc_tail_body(acc_ref, dinv_ref, b5_ref, wl_ref, bl_ref, out_ref, h_ref):
    dinv = dinv_ref[0].reshape(NP, 1)
    act = jnp.tanh(dinv * (acc_ref[0] + acc_ref[1]) + b5_ref[...])
    h_ref[...] = act[:N]
    out_ref[...] = jnp.dot(act[:N], wl_ref[...],
                           preferred_element_type=jnp.float32) + bl_ref[...]


_tc_tail = pl.pallas_call(
    _tc_tail_body,
    out_shape=[
        jax.ShapeDtypeStruct((N, 16), jnp.float32),
        jax.ShapeDtypeStruct((N, 2), jnp.float32),
    ],
)


def kernel(x, edge_index, W1, b1, W2, b2, W3, b3, W4, b4, W5, b5, Wl, bl):
    eiw = edge_index.astype(jnp.int32).reshape(2, NW, K, CH)
    ones_col = jnp.ones((NP, 1), jnp.float32)

    dacc = _sc_degree(ones_col, eiw)
    h1 = _tc_head(x, W1)
    acc, dinv2 = _sc_layer1(h1, dacc, eiw)
    acc = _make_sc_layer(16, 4)(acc, dinv2, W2, b1, eiw)
    acc = _make_sc_layer(4, 4)(acc, dinv2, W3, b2, eiw)
    acc = _make_sc_layer(4, 2)(acc, dinv2, W4, b3, eiw)
    acc = _make_sc_layer(2, 2)(acc, dinv2, W5, b4, eiw)
    out, h = _tc_tail(acc, dinv2, b5[None, :], Wl, bl[None, :])
    return (out, h)
